# Initial kernel scaffold; baseline (speedup 1.0000x reference)
import jax
import jax.numpy as jnp
from jax.experimental import pallas as pl


def kernel(x, edge_index, batch, W_conv, b_conv, w_ro, b_ro, W1, b1, W2, b2, W3, b3):
    raise NotImplementedError("write your pallas kernel here")


# speedup vs baseline: 6.5071x; 6.5071x over previous
import functools

import jax
import jax.numpy as jnp
from jax import lax
from jax.experimental import pallas as pl
from jax.experimental.pallas import tpu as pltpu
from jax.experimental.pallas import tpu_sc as plsc

_N = 10000
_E = 320000
_D = 128
_B = 64
_K = 3
_NC = 10

_NPAD = 10240
_NW = 32
_CHUNK = 128
_CH = 80
_EPW = _CH * _CHUNK
_EPAD = _NW * _EPW
_DUMMY_ROW = _N + 8
_RPT = _NPAD // 16
_BLK = 512
_NBLK = _NPAD // _BLK

_mesh = plsc.VectorSubcoreMesh(core_axis_name="c", subcore_axis_name="s")


def _sc_degree(dst3, ones16, zdeg):
    @functools.partial(
        pl.kernel, mesh=_mesh,
        out_type=jax.ShapeDtypeStruct((2, _NPAD, 16), jnp.float32),
        scratch_types=[
            pltpu.VMEM((_CH, _CHUNK), jnp.int32),
            pltpu.VMEM((_CHUNK, 16), jnp.float32),
            pltpu.VMEM_SHARED((_NPAD, 16), jnp.float32),
        ],
    )
    def k(dst_hbm, ones_hbm, z_hbm, out_hbm, idx_v, ones_v, deg_sp):
        c = lax.axis_index("c")
        s = lax.axis_index("s")
        wid = s * 2 + c
        pltpu.sync_copy(dst_hbm.at[wid], idx_v)
        pltpu.sync_copy(ones_hbm, ones_v)
        r0 = s * _RPT
        pltpu.sync_copy(z_hbm.at[pl.ds(r0, _RPT)], deg_sp.at[pl.ds(r0, _RPT)])
        plsc.subcore_barrier()

        @pl.loop(0, _CH)
        def _(j):
            pltpu.sync_copy(ones_v, deg_sp.at[idx_v.at[j]], add=True)

        plsc.subcore_barrier()
        pltpu.sync_copy(deg_sp.at[pl.ds(r0, _RPT)],
                        out_hbm.at[c, pl.ds(r0, _RPT)])

    return k(dst3, ones16, zdeg)


def _sc_propagate(g, src3, dst3, zfull):
    @functools.partial(
        pl.kernel, mesh=_mesh,
        out_type=jax.ShapeDtypeStruct((2, _NPAD, _D), jnp.float32),
        scratch_types=[
            pltpu.VMEM((_CH, _CHUNK), jnp.int32),
            pltpu.VMEM((_CH, _CHUNK), jnp.int32),
            pltpu.VMEM((_CHUNK, _D), jnp.float32),
            pltpu.VMEM_SHARED((_NPAD, _D), jnp.float32),
        ],
    )
    def k(g_hbm, s_hbm, d_hbm, z_hbm, out_hbm, si_v, di_v, buf, agg_sp):
        c = lax.axis_index("c")
        s = lax.axis_index("s")
        wid = s * 2 + c
        pltpu.sync_copy(s_hbm.at[wid], si_v)
        pltpu.sync_copy(d_hbm.at[wid], di_v)
        r0 = s * _RPT
        pltpu.sync_copy(z_hbm.at[pl.ds(r0, _RPT)], agg_sp.at[pl.ds(r0, _RPT)])
        plsc.subcore_barrier()

        @pl.loop(0, _CH)
        def _(j):
            pltpu.sync_copy(g_hbm.at[si_v.at[j]], buf)
            pltpu.sync_copy(buf, agg_sp.at[di_v.at[j]], add=True)

        plsc.subcore_barrier()
        pltpu.sync_copy(agg_sp.at[pl.ds(r0, _RPT)],
                        out_hbm.at[c, pl.ds(r0, _RPT)])

    return k(g, src3, dst3, zfull)


def _prep_body(degp_ref, x_ref, dis_ref, g_ref):
    dp = degp_ref[...]
    deg = dp[0, :, 0:1] + dp[1, :, 0:1] + 1.0
    dis = lax.rsqrt(deg)
    dis_ref[...] = dis
    g_ref[...] = x_ref[...] * dis


def _tc_prep(degp, xp):
    return pl.pallas_call(
        _prep_body,
        grid=(_NBLK,),
        in_specs=[
            pl.BlockSpec((2, _BLK, 16), lambda i: (0, i, 0)),
            pl.BlockSpec((_BLK, _D), lambda i: (i, 0)),
        ],
        out_specs=[
            pl.BlockSpec((_BLK, 1), lambda i: (i, 0)),
            pl.BlockSpec((_BLK, _D), lambda i: (i, 0)),
        ],
        out_shape=[
            jax.ShapeDtypeStruct((_NPAD, 1), jnp.float32),
            jax.ShapeDtypeStruct((_NPAD, _D), jnp.float32),
        ],
    )(degp, xp)


def _mix_body(t_ref, g_ref, h_ref, dis_ref, h1_ref, g1_ref):
    t = t_ref[...]
    dis = dis_ref[...]
    ssum = t[0] + t[1] + g_ref[...]
    h1 = 0.5 * (dis * ssum) + 0.5 * h_ref[...]
    h1_ref[...] = h1
    g1_ref[...] = dis * h1


def _tc_mix(t, g, h, dis):
    return pl.pallas_call(
        _mix_body,
        grid=(_NBLK,),
        in_specs=[
            pl.BlockSpec((2, _BLK, _D), lambda i: (0, i, 0)),
            pl.BlockSpec((_BLK, _D), lambda i: (i, 0)),
            pl.BlockSpec((_BLK, _D), lambda i: (i, 0)),
            pl.BlockSpec((_BLK, 1), lambda i: (i, 0)),
        ],
        out_specs=[
            pl.BlockSpec((_BLK, _D), lambda i: (i, 0)),
            pl.BlockSpec((_BLK, _D), lambda i: (i, 0)),
        ],
        out_shape=[
            jax.ShapeDtypeStruct((_NPAD, _D), jnp.float32),
            jax.ShapeDtypeStruct((_NPAD, _D), jnp.float32),
        ],
    )(t, g, h, dis)


def _pool_body(h_ref, b_ref, wc_ref, bc_ref, wro_ref, bro_ref,
               denom_ref, gsp_ref, gmp_ref):
    i = pl.program_id(0)

    @pl.when(i == 0)
    def _():
        denom_ref[...] = jnp.zeros_like(denom_ref)
        gsp_ref[...] = jnp.zeros_like(gsp_ref)
        gmp_ref[...] = jnp.full_like(gmp_ref, -1e30)

    h = h_ref[...]
    hc = jnp.dot(h, wc_ref[...], preferred_element_type=jnp.float32)
    hc = jnp.maximum(hc + bc_ref[...], 0.0)
    z = jnp.dot(hc, wro_ref[...], preferred_element_type=jnp.float32)
    z = z + bro_ref[...]
    v = 1.0 / (1.0 + jnp.exp(-z))
    vexp = jnp.exp(v)

    b = b_ref[...]
    seg = lax.broadcasted_iota(jnp.int32, (_BLK, _B), 1)
    msk = b == seg
    oh = msk.astype(jnp.float32)

    dims = (((0,), (0,)), ((), ()))
    denom_ref[...] += lax.dot_general(oh, vexp, dims,
                                      preferred_element_type=jnp.float32)
    gsp_ref[...] += lax.dot_general(oh, vexp * hc, dims,
                                    preferred_element_type=jnp.float32)

    def seg_body(s_i, _):
        m = b == s_i
        row = jnp.max(jnp.where(m, hc, -1e30), axis=0, keepdims=True)
        gmp_ref[pl.ds(s_i, 1), :] = jnp.maximum(gmp_ref[pl.ds(s_i, 1), :], row)
        return 0

    lax.fori_loop(0, _B, seg_body, 0)


def _tc_pool(hp, bp, W_conv, b_conv, w_ro, b_ro):
    return pl.pallas_call(
        _pool_body,
        grid=(_NBLK,),
        in_specs=[
            pl.BlockSpec((_BLK, _D), lambda i: (i, 0)),
            pl.BlockSpec((_BLK, 1), lambda i: (i, 0)),
            pl.BlockSpec((_D, _D), lambda i: (0, 0)),
            pl.BlockSpec((1, _D), lambda i: (0, 0)),
            pl.BlockSpec((_D, 1), lambda i: (0, 0)),
            pl.BlockSpec((1, 1), lambda i: (0, 0)),
        ],
        out_specs=[
            pl.BlockSpec((_B, 1), lambda i: (0, 0)),
            pl.BlockSpec((_B, _D), lambda i: (0, 0)),
            pl.BlockSpec((_B, _D), lambda i: (0, 0)),
        ],
        out_shape=[
            jax.ShapeDtypeStruct((_B, 1), jnp.float32),
            jax.ShapeDtypeStruct((_B, _D), jnp.float32),
            jax.ShapeDtypeStruct((_B, _D), jnp.float32),
        ],
    )(hp, bp, W_conv, b_conv, w_ro, b_ro)


def _head_body(denom_ref, gsp_ref, gmp_ref, w1a_ref, w1b_ref, b1_ref,
               w2_ref, b2_ref, w3_ref, b3_ref, out_ref):
    gsp = gsp_ref[...] / (denom_ref[...] + 1e-16)
    gmp = gmp_ref[...]
    o = jnp.dot(gmp, w1a_ref[...], preferred_element_type=jnp.float32)
    o = o + jnp.dot(gsp, w1b_ref[...], preferred_element_type=jnp.float32)
    o = jnp.maximum(o + b1_ref[...], 0.0)
    o = jnp.dot(o, w2_ref[...], preferred_element_type=jnp.float32)
    o = jnp.maximum(o + b2_ref[...], 0.0)
    o = jnp.dot(o, w3_ref[...], preferred_element_type=jnp.float32)
    o = o + b3_ref[...]
    m = jnp.max(o, axis=1, keepdims=True)
    e = jnp.exp(o - m)
    out_ref[...] = (o - m) - jnp.log(jnp.sum(e, axis=1, keepdims=True))


def _tc_head(denom, gsp_un, gmp, W1, b1, W2, b2, W3, b3):
    return pl.pallas_call(
        _head_body,
        out_shape=jax.ShapeDtypeStruct((_B, _NC), jnp.float32),
    )(denom, gsp_un, gmp, W1[:_D], W1[_D:], b1.reshape(1, -1),
      W2, b2.reshape(1, -1), W3, b3.reshape(1, -1))


@jax.jit
def _run(x, edge_index, batch, W_conv, b_conv, w_ro, b_ro,
         W1, b1, W2, b2, W3, b3):
    src = edge_index[0].astype(jnp.int32)
    dst = edge_index[1].astype(jnp.int32)
    epad_n = _EPAD - _E
    src3 = jnp.concatenate(
        [src, jnp.full((epad_n,), _DUMMY_ROW, jnp.int32)]).reshape(_NW, _CH, _CHUNK)
    dst3 = jnp.concatenate(
        [dst, jnp.full((epad_n,), _DUMMY_ROW, jnp.int32)]).reshape(_NW, _CH, _CHUNK)

    xp = jnp.pad(x, ((0, _NPAD - _N), (0, 0)))
    bp = jnp.pad(batch.astype(jnp.int32), (0, _NPAD - _N),
                 constant_values=_B).reshape(_NPAD, 1)

    ones16 = jnp.ones((_CHUNK, 16), jnp.float32)
    zdeg = jnp.zeros((_NPAD, 16), jnp.float32)
    zfull = jnp.zeros((_NPAD, _D), jnp.float32)

    degp = _sc_degree(dst3, ones16, zdeg)
    dis, g = _tc_prep(degp, xp)

    h = xp
    for _ in range(_K):
        t = _sc_propagate(g, src3, dst3, zfull)
        h, g = _tc_mix(t, g, h, dis)

    denom, gsp_un, gmp = _tc_pool(h, bp, W_conv, b_conv.reshape(1, _D),
                                  w_ro, b_ro.reshape(1, 1))
    return _tc_head(denom, gsp_un, gmp, W1, b1, W2, b2, W3, b3)


def kernel(x, edge_index, batch, W_conv, b_conv, w_ro, b_ro,
           W1, b1, W2, b2, W3, b3):
    return _run(x, edge_index, batch, W_conv, b_conv, w_ro, b_ro,
                W1, b1, W2, b2, W3, b3)


# speedup vs baseline: 15.2449x; 2.3428x over previous
import functools

import jax
import jax.numpy as jnp
from jax import lax
from jax.experimental import pallas as pl
from jax.experimental.pallas import tpu as pltpu
from jax.experimental.pallas import tpu_sc as plsc

_N = 10000
_E = 320000
_D = 128
_B = 64
_K = 3
_NC = 10

_NPAD = 10240
_NW = 32
_CHUNK = 128
_CH = 80
_EPW = _CH * _CHUNK
_EPAD = _NW * _EPW
_DUMMY_ROW = _N + 8
_RPT = _NPAD // 16
_ST = 16
_NST = _CH // _ST
_BLK = 512
_NBLK = _NPAD // _BLK

_mesh = plsc.VectorSubcoreMesh(core_axis_name="c", subcore_axis_name="s")


def _sc_degree(dst3, ones16, zdeg):
    @functools.partial(
        pl.kernel, mesh=_mesh,
        out_type=jax.ShapeDtypeStruct((2, _NPAD, 16), jnp.float32),
        scratch_types=[
            pltpu.VMEM((_CH, _CHUNK), jnp.int32),
            pltpu.VMEM((_CHUNK, 16), jnp.float32),
            pltpu.VMEM_SHARED((_NPAD, 16), jnp.float32),
        ],
    )
    def k(dst_hbm, ones_hbm, z_hbm, out_hbm, idx_v, ones_v, deg_sp):
        c = lax.axis_index("c")
        s = lax.axis_index("s")
        wid = s * 2 + c
        pltpu.sync_copy(dst_hbm.at[wid], idx_v)
        pltpu.sync_copy(ones_hbm, ones_v)
        r0 = s * _RPT
        pltpu.sync_copy(z_hbm.at[pl.ds(r0, _RPT)], deg_sp.at[pl.ds(r0, _RPT)])
        plsc.subcore_barrier()

        @pl.loop(0, _CH)
        def _(j):
            pltpu.sync_copy(ones_v, deg_sp.at[idx_v.at[j]], add=True)

        plsc.subcore_barrier()
        pltpu.sync_copy(deg_sp.at[pl.ds(r0, _RPT)],
                        out_hbm.at[c, pl.ds(r0, _RPT)])

    return k(dst3, ones16, zdeg)


def _sc_propagate(g, src3, dst3, zfull):
    @functools.partial(
        pl.kernel, mesh=_mesh,
        out_type=jax.ShapeDtypeStruct((2, _NPAD, _D), jnp.float32),
        scratch_types=[
            pltpu.VMEM((_ST, _CHUNK), jnp.int32),
            pltpu.VMEM((_ST, _CHUNK), jnp.int32),
            pltpu.VMEM((_CHUNK, _D), jnp.float32),
            pltpu.VMEM((_CHUNK, _D), jnp.float32),
            pltpu.VMEM_SHARED((_NPAD, _D), jnp.float32),
            pltpu.SemaphoreType.DMA,
        ],
    )
    def k(g_hbm, s_hbm, d_hbm, z_hbm, out_hbm, si_v, di_v, buf0, buf1,
          agg_sp, gsem):
        c = lax.axis_index("c")
        s = lax.axis_index("s")
        wid = s * 2 + c
        r0 = s * _RPT
        pltpu.sync_copy(z_hbm.at[pl.ds(r0, _RPT)], agg_sp.at[pl.ds(r0, _RPT)])
        plsc.subcore_barrier()

        @pl.loop(0, _NST)
        def _(st):
            pltpu.sync_copy(s_hbm.at[wid, pl.ds(st * _ST, _ST)], si_v)
            pltpu.sync_copy(d_hbm.at[wid, pl.ds(st * _ST, _ST)], di_v)
            pltpu.async_copy(g_hbm.at[si_v.at[0]], buf0, gsem)

            @pl.loop(0, _ST, step=2)
            def _(j):
                pltpu.make_async_copy(g_hbm.at[si_v.at[j]], buf0, gsem).wait()
                pltpu.async_copy(g_hbm.at[si_v.at[j + 1]], buf1, gsem)
                pltpu.sync_copy(buf0, agg_sp.at[di_v.at[j]], add=True)
                pltpu.make_async_copy(g_hbm.at[si_v.at[j + 1]], buf1,
                                      gsem).wait()

                @pl.when(j + 2 < _ST)
                def _():
                    pltpu.async_copy(g_hbm.at[si_v.at[j + 2]], buf0, gsem)

                pltpu.sync_copy(buf1, agg_sp.at[di_v.at[j + 1]], add=True)

        plsc.subcore_barrier()
        pltpu.sync_copy(agg_sp.at[pl.ds(r0, _RPT)],
                        out_hbm.at[c, pl.ds(r0, _RPT)])

    return k(g, src3, dst3, zfull)


def _prep_body(degp_ref, x_ref, dis_ref, g_ref):
    dp = degp_ref[...]
    deg = dp[0, :, 0:1] + dp[1, :, 0:1] + 1.0
    dis = lax.rsqrt(deg)
    dis_ref[...] = dis
    g_ref[...] = x_ref[...] * dis


def _tc_prep(degp, xp):
    return pl.pallas_call(
        _prep_body,
        grid=(_NBLK,),
        in_specs=[
            pl.BlockSpec((2, _BLK, 16), lambda i: (0, i, 0)),
            pl.BlockSpec((_BLK, _D), lambda i: (i, 0)),
        ],
        out_specs=[
            pl.BlockSpec((_BLK, 1), lambda i: (i, 0)),
            pl.BlockSpec((_BLK, _D), lambda i: (i, 0)),
        ],
        out_shape=[
            jax.ShapeDtypeStruct((_NPAD, 1), jnp.float32),
            jax.ShapeDtypeStruct((_NPAD, _D), jnp.float32),
        ],
    )(degp, xp)


def _mix_body(t_ref, g_ref, h_ref, dis_ref, h1_ref, g1_ref):
    t = t_ref[...]
    dis = dis_ref[...]
    ssum = t[0] + t[1] + g_ref[...]
    h1 = 0.5 * (dis * ssum) + 0.5 * h_ref[...]
    h1_ref[...] = h1
    g1_ref[...] = dis * h1


def _tc_mix(t, g, h, dis):
    return pl.pallas_call(
        _mix_body,
        grid=(_NBLK,),
        in_specs=[
            pl.BlockSpec((2, _BLK, _D), lambda i: (0, i, 0)),
            pl.BlockSpec((_BLK, _D), lambda i: (i, 0)),
            pl.BlockSpec((_BLK, _D), lambda i: (i, 0)),
            pl.BlockSpec((_BLK, 1), lambda i: (i, 0)),
        ],
        out_specs=[
            pl.BlockSpec((_BLK, _D), lambda i: (i, 0)),
            pl.BlockSpec((_BLK, _D), lambda i: (i, 0)),
        ],
        out_shape=[
            jax.ShapeDtypeStruct((_NPAD, _D), jnp.float32),
            jax.ShapeDtypeStruct((_NPAD, _D), jnp.float32),
        ],
    )(t, g, h, dis)


def _pool_body(h_ref, b_ref, wc_ref, bc_ref, wro_ref, bro_ref,
               denom_ref, gsp_ref, gmp_ref):
    i = pl.program_id(0)

    @pl.when(i == 0)
    def _():
        denom_ref[...] = jnp.zeros_like(denom_ref)
        gsp_ref[...] = jnp.zeros_like(gsp_ref)
        gmp_ref[...] = jnp.full_like(gmp_ref, -1e30)

    h = h_ref[...]
    hc = jnp.dot(h, wc_ref[...], preferred_element_type=jnp.float32)
    hc = jnp.maximum(hc + bc_ref[...], 0.0)
    z = jnp.dot(hc, wro_ref[...], preferred_element_type=jnp.float32)
    z = z + bro_ref[...]
    v = 1.0 / (1.0 + jnp.exp(-z))
    vexp = jnp.exp(v)

    b = b_ref[...]
    seg = lax.broadcasted_iota(jnp.int32, (_BLK, _B), 1)
    msk = b == seg
    oh = msk.astype(jnp.float32)

    dims = (((0,), (0,)), ((), ()))
    denom_ref[...] += lax.dot_general(oh, vexp, dims,
                                      preferred_element_type=jnp.float32)
    gsp_ref[...] += lax.dot_general(oh, vexp * hc, dims,
                                    preferred_element_type=jnp.float32)

    def seg_body(s_i, _):
        m = b == s_i
        row = jnp.max(jnp.where(m, hc, -1e30), axis=0, keepdims=True)
        gmp_ref[pl.ds(s_i, 1), :] = jnp.maximum(gmp_ref[pl.ds(s_i, 1), :], row)
        return 0

    lax.fori_loop(0, _B, seg_body, 0)


def _tc_pool(hp, bp, W_conv, b_conv, w_ro, b_ro):
    return pl.pallas_call(
        _pool_body,
        grid=(_NBLK,),
        in_specs=[
            pl.BlockSpec((_BLK, _D), lambda i: (i, 0)),
            pl.BlockSpec((_BLK, 1), lambda i: (i, 0)),
            pl.BlockSpec((_D, _D), lambda i: (0, 0)),
            pl.BlockSpec((1, _D), lambda i: (0, 0)),
            pl.BlockSpec((_D, 1), lambda i: (0, 0)),
            pl.BlockSpec((1, 1), lambda i: (0, 0)),
        ],
        out_specs=[
            pl.BlockSpec((_B, 1), lambda i: (0, 0)),
            pl.BlockSpec((_B, _D), lambda i: (0, 0)),
            pl.BlockSpec((_B, _D), lambda i: (0, 0)),
        ],
        out_shape=[
            jax.ShapeDtypeStruct((_B, 1), jnp.float32),
            jax.ShapeDtypeStruct((_B, _D), jnp.float32),
            jax.ShapeDtypeStruct((_B, _D), jnp.float32),
        ],
    )(hp, bp, W_conv, b_conv, w_ro, b_ro)


def _head_body(denom_ref, gsp_ref, gmp_ref, w1a_ref, w1b_ref, b1_ref,
               w2_ref, b2_ref, w3_ref, b3_ref, out_ref):
    gsp = gsp_ref[...] / (denom_ref[...] + 1e-16)
    gmp = gmp_ref[...]
    o = jnp.dot(gmp, w1a_ref[...], preferred_element_type=jnp.float32)
    o = o + jnp.dot(gsp, w1b_ref[...], preferred_element_type=jnp.float32)
    o = jnp.maximum(o + b1_ref[...], 0.0)
    o = jnp.dot(o, w2_ref[...], preferred_element_type=jnp.float32)
    o = jnp.maximum(o + b2_ref[...], 0.0)
    o = jnp.dot(o, w3_ref[...], preferred_element_type=jnp.float32)
    o = o + b3_ref[...]
    m = jnp.max(o, axis=1, keepdims=True)
    e = jnp.exp(o - m)
    out_ref[...] = (o - m) - jnp.log(jnp.sum(e, axis=1, keepdims=True))


def _tc_head(denom, gsp_un, gmp, W1, b1, W2, b2, W3, b3):
    return pl.pallas_call(
        _head_body,
        out_shape=jax.ShapeDtypeStruct((_B, _NC), jnp.float32),
    )(denom, gsp_un, gmp, W1[:_D], W1[_D:], b1.reshape(1, -1),
      W2, b2.reshape(1, -1), W3, b3.reshape(1, -1))


@jax.jit
def _run(x, edge_index, batch, W_conv, b_conv, w_ro, b_ro,
         W1, b1, W2, b2, W3, b3):
    src = edge_index[0].astype(jnp.int32)
    dst = edge_index[1].astype(jnp.int32)
    epad_n = _EPAD - _E
    pad_rows = _N + (jnp.arange(epad_n, dtype=jnp.int32) % (_NPAD - _N))
    src3 = jnp.concatenate([src, pad_rows]).reshape(_NW, _CH, _CHUNK)
    dst3 = jnp.concatenate([dst, pad_rows]).reshape(_NW, _CH, _CHUNK)

    xp = jnp.pad(x, ((0, _NPAD - _N), (0, 0)))
    bp = jnp.pad(batch.astype(jnp.int32), (0, _NPAD - _N),
                 constant_values=_B).reshape(_NPAD, 1)

    ones16 = jnp.ones((_CHUNK, 16), jnp.float32)
    zdeg = jnp.zeros((_NPAD, 16), jnp.float32)
    zfull = jnp.zeros((_NPAD, _D), jnp.float32)

    degp = _sc_degree(dst3, ones16, zdeg)
    dis, g = _tc_prep(degp, xp)

    h = xp
    for _ in range(_K):
        t = _sc_propagate(g, src3, dst3, zfull)
        h, g = _tc_mix(t, g, h, dis)

    denom, gsp_un, gmp = _tc_pool(h, bp, W_conv, b_conv.reshape(1, _D),
                                  w_ro, b_ro.reshape(1, 1))
    return _tc_head(denom, gsp_un, gmp, W1, b1, W2, b2, W3, b3)


def kernel(x, edge_index, batch, W_conv, b_conv, w_ro, b_ro,
           W1, b1, W2, b2, W3, b3):
    return _run(x, edge_index, batch, W_conv, b_conv, w_ro, b_ro,
                W1, b1, W2, b2, W3, b3)
